# baseline (speedup 1.0000x reference)
import jax
import jax.numpy as jnp
from jax import lax
from jax.experimental import pallas as pl
from jax.experimental.pallas import tpu as pltpu
from jax.experimental.pallas import tpu_sc as plsc

NW = 32
B = 16384
D = 64
L = 16
NU = 1000000
NBLK = (NU + 127) // 128
BPWK = (NBLK + NW - 1) // NW
CHB = 2
NSLOT = 4
CHU = CHB * 128
PBITS = 15
PMASK = (1 << PBITS) - 1
ROWS_PAD = B + NW
NPIECE = 4
PIECE = B // NPIECE


def _scan_table(tab_hbm, ids_row, rows_hbm, wid, gs,
                piece_v, hits_v, ch_v, chunk_v, stage_v,
                sem_ids, sem_chk, sem_sc):
    blk0 = wid * BPWK
    u_lo = blk0 * 128
    nb = jnp.minimum(BPWK, NBLK - blk0)
    u_hi = jnp.minimum(u_lo + nb * 128, NU)
    trash = B + wid

    for pc in range(2):
        pltpu.async_copy(ids_row.at[pl.ds(pc * PIECE, PIECE)],
                         piece_v.at[pc], sem_ids[pc])

    nh = 0
    for pc in range(NPIECE):
        pltpu.make_async_copy(ids_row.at[pl.ds(0, PIECE)],
                              piece_v.at[pc & 1], sem_ids[pc & 1]).wait()

        def scan(g, off, _pc=pc):
            v = piece_v[_pc & 1, pl.ds(g * L, L)]
            p = _pc * PIECE + g * L + lax.iota(jnp.int32, L)
            m = (v >= u_lo) & (v < u_hi)
            pk = ((v - u_lo) << PBITS) | p
            plsc.store_compressed(hits_v.at[pl.ds(off, L)], pk, mask=m)
            return off + plsc.all_reduce_population_count(m)[0]

        nh = lax.fori_loop(0, PIECE // L, scan, nh)
        if pc + 2 < NPIECE:
            pltpu.async_copy(ids_row.at[pl.ds((pc + 2) * PIECE, PIECE)],
                             piece_v.at[pc & 1], sem_ids[pc & 1])

    nch = (nb + CHB - 1) // CHB

    def fire_chunk(c):
        @pl.when(c < nch)
        def _():
            for b in range(NSLOT):
                @pl.when((c & 3) == b)
                def _(b=b):
                    for cb in range(CHB):
                        start = pl.multiple_of(
                            u_lo + c * CHU + cb * 128, 128)
                        pltpu.async_copy(tab_hbm.at[:, pl.ds(start, 128)],
                                         chunk_v.at[b, cb], sem_chk[b])

    for cpre in range(NSLOT - 1):
        fire_chunk(jnp.int32(cpre))

    def chunk_body(c, gs_c):
        slot = c & 3
        for b in range(NSLOT):
            @pl.when((c & 3) == b)
            def _(b=b):
                for cb in range(CHB):
                    pltpu.make_async_copy(tab_hbm.at[:, pl.ds(0, 128)],
                                          chunk_v.at[b, cb],
                                          sem_chk[b]).wait()
        fire_chunk(c + NSLOT - 1)

        clo = c * CHU

        def comp(hv, off2):
            o = pl.multiple_of(hv * L, L)
            pk = hits_v[pl.ds(o, L)]
            lane_ok = (hv * L + lax.iota(jnp.int32, L)) < nh
            rel = pk >> PBITS
            m2 = (rel >= clo) & (rel < clo + CHU) & lane_ok
            plsc.store_compressed(ch_v.at[pl.ds(off2, L)], pk, mask=m2)
            return off2 + plsc.all_reduce_population_count(m2)[0]

        nh2 = lax.fori_loop(0, (nh + L - 1) // L, comp, 0)
        pad = jnp.full((L,), (clo << PBITS) | trash, jnp.int32)
        plsc.store_compressed(ch_v.at[pl.ds(nh2, L)], pad,
                              mask=jnp.full((L,), True, jnp.bool_))

        ns = (nh2 + L - 1) >> 4

        def shot(s, gs_s):
            for b in range(4):
                @pl.when(((gs_s & 3) == b) & (gs_s >= 4))
                def _(b=b):
                    pltpu.make_async_copy(rows_hbm.at[pl.ds(0, L)],
                                          stage_v.at[b], sem_sc[b]).wait()

            buf = gs_s & 3
            pk16 = ch_v[pl.ds(pl.multiple_of(s * L, L), L)]
            p16 = pk16 & PMASK
            rel = (pk16 >> PBITS) - clo
            blk = rel >> 7
            lane = rel & 127
            stage_s = stage_v.at[buf]
            rows16 = lax.iota(jnp.int32, L)
            slot16 = jnp.full((L,), slot, jnp.int32)
            for j in range(D):
                j16 = jnp.full((L,), j, jnp.int32)
                val = plsc.load_gather(chunk_v, [slot16, blk, j16, lane])
                plsc.store_scatter(stage_s, [rows16, j16], val)
            for b in range(4):
                @pl.when((gs_s & 3) == b)
                def _(b=b):
                    pltpu.async_copy(stage_v.at[b], rows_hbm.at[p16],
                                     sem_sc[b])
            return gs_s + 1

        return lax.fori_loop(0, ns, shot, gs_c)

    return lax.fori_loop(0, nch, chunk_body, gs)


def _gather_body(uid_hbm, iid_hbm, uf_hbm, if_hbm, u_rows, i_rows,
                 piece_v, hits_v, ch_v, chunk_v, stage_v,
                 sem_ids0, sem_ids1,
                 sem_chk0, sem_chk1, sem_chk2, sem_chk3,
                 sem_sc0, sem_sc1, sem_sc2, sem_sc3):
    c_id = lax.axis_index("c")
    s_id = lax.axis_index("s")
    wid = s_id * 2 + c_id
    sem_ids = (sem_ids0, sem_ids1)
    sem_chk = (sem_chk0, sem_chk1, sem_chk2, sem_chk3)
    sem_sc = (sem_sc0, sem_sc1, sem_sc2, sem_sc3)

    gs = _scan_table(uf_hbm, uid_hbm, u_rows, wid, 0,
                     piece_v, hits_v, ch_v, chunk_v, stage_v,
                     sem_ids, sem_chk, sem_sc)
    gs = _scan_table(if_hbm, iid_hbm, i_rows, wid, gs,
                     piece_v, hits_v, ch_v, chunk_v, stage_v,
                     sem_ids, sem_chk, sem_sc)

    for b in range(4):
        @pl.when(gs > b)
        def _(b=b):
            pltpu.make_async_copy(u_rows.at[pl.ds(0, L)],
                                  stage_v.at[b], sem_sc[b]).wait()


BPQ = B // NW
HPQ = BPQ // 2


def _dot_body(u_rows, i_rows, out_hbm, u_v, i_v, out_v, sem):
    c_id = lax.axis_index("c")
    s_id = lax.axis_index("s")
    wid = s_id * 2 + c_id

    for h in range(2):
        base = wid * BPQ + h * HPQ
        cu = pltpu.async_copy(u_rows.at[pl.ds(base, HPQ)], u_v, sem)
        ci = pltpu.async_copy(i_rows.at[pl.ds(base, HPQ)], i_v, sem)
        cu.wait()
        ci.wait()

        def group(g, carry):
            q16 = g * L + lax.iota(jnp.int32, L)
            acc = jnp.zeros((L,), jnp.float32)
            for j in range(D):
                j16 = jnp.full((L,), j, jnp.int32)
                u = plsc.load_gather(u_v, [q16, j16])
                v = plsc.load_gather(i_v, [q16, j16])
                acc = acc + u * v
            out_v[pl.ds(g * L, L)] = acc
            return carry

        lax.fori_loop(0, HPQ // L, group, 0)
        pltpu.sync_copy(out_v, out_hbm.at[pl.ds(base, HPQ)])


@jax.jit
def kernel(user_item_tuple, user_factors, item_factors):
    ids = user_item_tuple.astype(jnp.int32)
    uid = ids[:, 0]
    iid = ids[:, 1]
    uf_t = user_factors.T
    if_t = item_factors.T
    mesh = plsc.VectorSubcoreMesh(core_axis_name="c", subcore_axis_name="s")
    params = pltpu.CompilerParams(
        needs_layout_passes=False, use_tc_tiling_on_sc=True)

    gather = pl.kernel(
        _gather_body,
        out_type=(jax.ShapeDtypeStruct((ROWS_PAD, 128), jnp.float32),
                  jax.ShapeDtypeStruct((ROWS_PAD, 128), jnp.float32)),
        mesh=mesh,
        scratch_types=[
            pltpu.VMEM((2, PIECE), jnp.int32),
            pltpu.VMEM((B + L,), jnp.int32),
            pltpu.VMEM((B + L,), jnp.int32),
            pltpu.VMEM((NSLOT, CHB, D, 128), jnp.float32),
            pltpu.VMEM((4, L, 128), jnp.float32),
            pltpu.SemaphoreType.DMA,
            pltpu.SemaphoreType.DMA,
            pltpu.SemaphoreType.DMA,
            pltpu.SemaphoreType.DMA,
            pltpu.SemaphoreType.DMA,
            pltpu.SemaphoreType.DMA,
            pltpu.SemaphoreType.DMA,
            pltpu.SemaphoreType.DMA,
            pltpu.SemaphoreType.DMA,
            pltpu.SemaphoreType.DMA,
        ],
        compiler_params=params,
    )
    u_rows, i_rows = gather(uid, iid, uf_t, if_t)

    dot = pl.kernel(
        _dot_body,
        out_type=jax.ShapeDtypeStruct((B,), jnp.float32),
        mesh=mesh,
        scratch_types=[
            pltpu.VMEM((HPQ, 128), jnp.float32),
            pltpu.VMEM((HPQ, 128), jnp.float32),
            pltpu.VMEM((HPQ,), jnp.float32),
            pltpu.SemaphoreType.DMA,
        ],
        compiler_params=params,
    )
    return dot(u_rows, i_rows)


# speedup vs baseline: 1.4641x; 1.4641x over previous
import jax
import jax.numpy as jnp
from jax import lax
from jax.experimental import pallas as pl
from jax.experimental.pallas import tpu as pltpu
from jax.experimental.pallas import tpu_sc as plsc

NW = 32
B = 16384
D = 64
L = 16
NU = 1000000
NBLK = (NU + 127) // 128
BPWK = (NBLK + NW - 1) // NW
CHB = 4
CHU = CHB * 128
PBITS = 15
PMASK = (1 << PBITS) - 1
ROWS_PAD = B + NW
NPIECE = 4
PIECE = B // NPIECE


def _scan_table(tab_hbm, ids_row, rows_hbm, wid, gs,
                piece_v, hits_v, ch_v, chunk_v, stage_v,
                sem_ids, sem_chk, sem_sc):
    blk0 = wid * BPWK
    u_lo = blk0 * 128
    nb = jnp.minimum(BPWK, NBLK - blk0)
    u_hi = jnp.minimum(u_lo + nb * 128, NU)
    trash = B + wid

    for pc in range(2):
        pltpu.async_copy(ids_row.at[pl.ds(pc * PIECE, PIECE)],
                         piece_v.at[pc], sem_ids[pc])

    nh = 0
    for pc in range(NPIECE):
        pltpu.make_async_copy(ids_row.at[pl.ds(0, PIECE)],
                              piece_v.at[pc & 1], sem_ids[pc & 1]).wait()

        def scan(g, off, _pc=pc):
            v = piece_v[_pc & 1, pl.ds(g * L, L)]
            p = _pc * PIECE + g * L + lax.iota(jnp.int32, L)
            m = (v >= u_lo) & (v < u_hi)
            pk = ((v - u_lo) << PBITS) | p
            plsc.store_compressed(hits_v.at[pl.ds(off, L)], pk, mask=m)
            return off + plsc.all_reduce_population_count(m)[0]

        nh = lax.fori_loop(0, PIECE // L, scan, nh)
        if pc + 2 < NPIECE:
            pltpu.async_copy(ids_row.at[pl.ds((pc + 2) * PIECE, PIECE)],
                             piece_v.at[pc & 1], sem_ids[pc & 1])

    nch = (nb + CHB - 1) // CHB

    def fire_chunk(c, slot):
        @pl.when(c < nch)
        def _():
            for cb in range(CHB):
                start = pl.multiple_of(u_lo + c * CHU + cb * 128, 128)
                pltpu.async_copy(tab_hbm.at[:, pl.ds(start, 128)],
                                 chunk_v.at[slot, cb], sem_chk[0])

    fire_chunk(0, 0)

    def chunk_body(c, gs_c):
        slot = c & 1
        for cb in range(CHB):
            pltpu.make_async_copy(tab_hbm.at[:, pl.ds(0, 128)],
                                  chunk_v.at[slot, cb], sem_chk[0]).wait()
        fire_chunk(c + 1, slot ^ 1)

        clo = c * CHU

        def comp(hv, off2):
            o = pl.multiple_of(hv * L, L)
            pk = hits_v[pl.ds(o, L)]
            lane_ok = (hv * L + lax.iota(jnp.int32, L)) < nh
            rel = pk >> PBITS
            m2 = (rel >= clo) & (rel < clo + CHU) & lane_ok
            plsc.store_compressed(ch_v.at[pl.ds(off2, L)], pk, mask=m2)
            return off2 + plsc.all_reduce_population_count(m2)[0]

        nh2 = lax.fori_loop(0, (nh + L - 1) // L, comp, 0)
        pad = jnp.full((L,), (clo << PBITS) | trash, jnp.int32)
        plsc.store_compressed(ch_v.at[pl.ds(nh2, L)], pad,
                              mask=jnp.full((L,), True, jnp.bool_))

        ns = (nh2 + L - 1) >> 4

        def shot(s, gs_s):
            for b in range(4):
                @pl.when(((gs_s & 3) == b) & (gs_s >= 4))
                def _(b=b):
                    pltpu.make_async_copy(rows_hbm.at[pl.ds(0, L)],
                                          stage_v.at[b], sem_sc[b]).wait()

            buf = gs_s & 3
            pk16 = ch_v[pl.ds(pl.multiple_of(s * L, L), L)]
            p16 = pk16 & PMASK
            rel = (pk16 >> PBITS) - clo
            blk = rel >> 7
            lane = rel & 127
            stage_s = stage_v.at[buf]
            rows16 = lax.iota(jnp.int32, L)
            slot16 = jnp.full((L,), slot, jnp.int32)
            for j in range(D):
                j16 = jnp.full((L,), j, jnp.int32)
                val = plsc.load_gather(chunk_v, [slot16, blk, j16, lane])
                plsc.store_scatter(stage_s, [rows16, j16], val)
            for b in range(4):
                @pl.when((gs_s & 3) == b)
                def _(b=b):
                    pltpu.async_copy(stage_v.at[b], rows_hbm.at[p16],
                                     sem_sc[b])
            return gs_s + 1

        return lax.fori_loop(0, ns, shot, gs_c)

    return lax.fori_loop(0, nch, chunk_body, gs)


def _gather_body(uid_hbm, iid_hbm, uf_hbm, if_hbm, u_rows, i_rows,
                 piece_v, hits_v, ch_v, chunk_v, stage_v,
                 sem_ids0, sem_ids1,
                 sem_chk0, sem_chk1, sem_chk2, sem_chk3,
                 sem_sc0, sem_sc1, sem_sc2, sem_sc3):
    c_id = lax.axis_index("c")
    s_id = lax.axis_index("s")
    wid = s_id * 2 + c_id
    sem_ids = (sem_ids0, sem_ids1)
    sem_chk = (sem_chk0, sem_chk1, sem_chk2, sem_chk3)
    sem_sc = (sem_sc0, sem_sc1, sem_sc2, sem_sc3)

    gs = _scan_table(uf_hbm, uid_hbm, u_rows, wid, 0,
                     piece_v, hits_v, ch_v, chunk_v, stage_v,
                     sem_ids, sem_chk, sem_sc)
    gs = _scan_table(if_hbm, iid_hbm, i_rows, wid, gs,
                     piece_v, hits_v, ch_v, chunk_v, stage_v,
                     sem_ids, sem_chk, sem_sc)

    for b in range(4):
        @pl.when(gs > b)
        def _(b=b):
            pltpu.make_async_copy(u_rows.at[pl.ds(0, L)],
                                  stage_v.at[b], sem_sc[b]).wait()


BPQ = B // NW
HPQ = BPQ // 2


def _dot_body(u_rows, i_rows, out_hbm, u_v, i_v, out_v, sem):
    c_id = lax.axis_index("c")
    s_id = lax.axis_index("s")
    wid = s_id * 2 + c_id

    for h in range(2):
        base = wid * BPQ + h * HPQ
        cu = pltpu.async_copy(u_rows.at[pl.ds(base, HPQ)], u_v, sem)
        ci = pltpu.async_copy(i_rows.at[pl.ds(base, HPQ)], i_v, sem)
        cu.wait()
        ci.wait()

        def group(g, carry):
            q16 = g * L + lax.iota(jnp.int32, L)
            acc = jnp.zeros((L,), jnp.float32)
            for j in range(D):
                j16 = jnp.full((L,), j, jnp.int32)
                u = plsc.load_gather(u_v, [q16, j16])
                v = plsc.load_gather(i_v, [q16, j16])
                acc = acc + u * v
            out_v[pl.ds(g * L, L)] = acc
            return carry

        lax.fori_loop(0, HPQ // L, group, 0)
        pltpu.sync_copy(out_v, out_hbm.at[pl.ds(base, HPQ)])


@jax.jit
def kernel(user_item_tuple, user_factors, item_factors):
    ids = user_item_tuple.astype(jnp.int32)
    uid = ids[:, 0]
    iid = ids[:, 1]
    uf_t = user_factors.T
    if_t = item_factors.T
    mesh = plsc.VectorSubcoreMesh(core_axis_name="c", subcore_axis_name="s")
    params = pltpu.CompilerParams(
        needs_layout_passes=False, use_tc_tiling_on_sc=True)

    gather = pl.kernel(
        _gather_body,
        out_type=(jax.ShapeDtypeStruct((ROWS_PAD, 128), jnp.float32),
                  jax.ShapeDtypeStruct((ROWS_PAD, 128), jnp.float32)),
        mesh=mesh,
        scratch_types=[
            pltpu.VMEM((2, PIECE), jnp.int32),
            pltpu.VMEM((B + L,), jnp.int32),
            pltpu.VMEM((B + L,), jnp.int32),
            pltpu.VMEM((2, CHB, D, 128), jnp.float32),
            pltpu.VMEM((4, L, 128), jnp.float32),
            pltpu.SemaphoreType.DMA,
            pltpu.SemaphoreType.DMA,
            pltpu.SemaphoreType.DMA,
            pltpu.SemaphoreType.DMA,
            pltpu.SemaphoreType.DMA,
            pltpu.SemaphoreType.DMA,
            pltpu.SemaphoreType.DMA,
            pltpu.SemaphoreType.DMA,
            pltpu.SemaphoreType.DMA,
            pltpu.SemaphoreType.DMA,
        ],
        compiler_params=params,
    )
    u_rows, i_rows = gather(uid, iid, uf_t, if_t)

    dot = pl.kernel(
        _dot_body,
        out_type=jax.ShapeDtypeStruct((B,), jnp.float32),
        mesh=mesh,
        scratch_types=[
            pltpu.VMEM((HPQ, 128), jnp.float32),
            pltpu.VMEM((HPQ, 128), jnp.float32),
            pltpu.VMEM((HPQ,), jnp.float32),
            pltpu.SemaphoreType.DMA,
        ],
        compiler_params=params,
    )
    return dot(u_rows, i_rows)


# speedup vs baseline: 1.4652x; 1.0008x over previous
import jax
import jax.numpy as jnp
from jax import lax
from jax.experimental import pallas as pl
from jax.experimental.pallas import tpu as pltpu
from jax.experimental.pallas import tpu_sc as plsc

NW = 32
B = 16384
D = 64
L = 16
NU = 1000000
NBLK = (NU + 127) // 128
BPWK = (NBLK + NW - 1) // NW
CHB = 4
CHU = CHB * 128
PBITS = 15
PMASK = (1 << PBITS) - 1
ROWS_PAD = B + NW
NPIECE = 4
PIECE = B // NPIECE


def _scan_table(tab_hbm, ids_row, rows_hbm, wid, gs,
                piece_v, hits_v, ch_v, chunk_v, stage_v,
                sem_ids, sem_chk, sem_sc):
    blk0 = wid * BPWK
    u_lo = blk0 * 128
    nb = jnp.minimum(BPWK, NBLK - blk0)
    u_hi = jnp.minimum(u_lo + nb * 128, NU)
    trash = B + wid

    for pc in range(2):
        pltpu.async_copy(ids_row.at[pl.ds(pc * PIECE, PIECE)],
                         piece_v.at[pc], sem_ids[pc])

    nh = 0
    for pc in range(NPIECE):
        pltpu.make_async_copy(ids_row.at[pl.ds(0, PIECE)],
                              piece_v.at[pc & 1], sem_ids[pc & 1]).wait()

        def scan(g, off, _pc=pc):
            v = piece_v[_pc & 1, pl.ds(g * L, L)]
            p = _pc * PIECE + g * L + lax.iota(jnp.int32, L)
            m = (v >= u_lo) & (v < u_hi)
            pk = ((v - u_lo) << PBITS) | p
            plsc.store_compressed(hits_v.at[pl.ds(off, L)], pk, mask=m)
            return off + plsc.all_reduce_population_count(m)[0]

        nh = lax.fori_loop(0, PIECE // L, scan, nh)
        if pc + 2 < NPIECE:
            pltpu.async_copy(ids_row.at[pl.ds((pc + 2) * PIECE, PIECE)],
                             piece_v.at[pc & 1], sem_ids[pc & 1])

    nch = (nb + CHB - 1) // CHB

    def fire_chunk(c, slot):
        @pl.when(c < nch)
        def _():
            start = pl.multiple_of(u_lo + c * CHU, 128)
            pltpu.async_copy(tab_hbm.at[:, pl.ds(start, CHU)],
                             chunk_v.at[slot], sem_chk[0])

    fire_chunk(0, 0)

    def chunk_body(c, gs_c):
        slot = c & 1
        pltpu.make_async_copy(tab_hbm.at[:, pl.ds(0, CHU)],
                              chunk_v.at[slot], sem_chk[0]).wait()
        fire_chunk(c + 1, slot ^ 1)

        clo = c * CHU

        def comp(hv, off2):
            o = pl.multiple_of(hv * L, L)
            pk = hits_v[pl.ds(o, L)]
            lane_ok = (hv * L + lax.iota(jnp.int32, L)) < nh
            rel = pk >> PBITS
            m2 = (rel >= clo) & (rel < clo + CHU) & lane_ok
            plsc.store_compressed(ch_v.at[pl.ds(off2, L)], pk, mask=m2)
            return off2 + plsc.all_reduce_population_count(m2)[0]

        nh2 = lax.fori_loop(0, (nh + L - 1) // L, comp, 0)
        pad = jnp.full((L,), (clo << PBITS) | trash, jnp.int32)
        plsc.store_compressed(ch_v.at[pl.ds(nh2, L)], pad,
                              mask=jnp.full((L,), True, jnp.bool_))

        ns = (nh2 + L - 1) >> 4

        def shot(s, gs_s):
            for b in range(4):
                @pl.when(((gs_s & 3) == b) & (gs_s >= 4))
                def _(b=b):
                    pltpu.make_async_copy(rows_hbm.at[pl.ds(0, L)],
                                          stage_v.at[b], sem_sc[b]).wait()

            buf = gs_s & 3
            pk16 = ch_v[pl.ds(pl.multiple_of(s * L, L), L)]
            p16 = pk16 & PMASK
            rel = (pk16 >> PBITS) - clo
            stage_s = stage_v.at[buf]
            rows16 = lax.iota(jnp.int32, L)
            slot16 = jnp.full((L,), slot, jnp.int32)
            for j in range(D):
                j16 = jnp.full((L,), j, jnp.int32)
                val = plsc.load_gather(chunk_v, [slot16, j16, rel])
                plsc.store_scatter(stage_s, [rows16, j16], val)
            for b in range(4):
                @pl.when((gs_s & 3) == b)
                def _(b=b):
                    pltpu.async_copy(stage_v.at[b], rows_hbm.at[p16],
                                     sem_sc[b])
            return gs_s + 1

        return lax.fori_loop(0, ns, shot, gs_c)

    return lax.fori_loop(0, nch, chunk_body, gs)


def _gather_body(uid_hbm, iid_hbm, uf_hbm, if_hbm, u_rows, i_rows,
                 piece_v, hits_v, ch_v, chunk_v, stage_v,
                 sem_ids0, sem_ids1,
                 sem_chk0, sem_chk1, sem_chk2, sem_chk3,
                 sem_sc0, sem_sc1, sem_sc2, sem_sc3):
    c_id = lax.axis_index("c")
    s_id = lax.axis_index("s")
    wid = s_id * 2 + c_id
    sem_ids = (sem_ids0, sem_ids1)
    sem_chk = (sem_chk0, sem_chk1, sem_chk2, sem_chk3)
    sem_sc = (sem_sc0, sem_sc1, sem_sc2, sem_sc3)

    gs = _scan_table(uf_hbm, uid_hbm, u_rows, wid, 0,
                     piece_v, hits_v, ch_v, chunk_v, stage_v,
                     sem_ids, sem_chk, sem_sc)
    gs = _scan_table(if_hbm, iid_hbm, i_rows, wid, gs,
                     piece_v, hits_v, ch_v, chunk_v, stage_v,
                     sem_ids, sem_chk, sem_sc)

    for b in range(4):
        @pl.when(gs > b)
        def _(b=b):
            pltpu.make_async_copy(u_rows.at[pl.ds(0, L)],
                                  stage_v.at[b], sem_sc[b]).wait()


BPQ = B // NW
HPQ = BPQ // 2


def _dot_body(u_rows, i_rows, out_hbm, u_v, i_v, out_v, sem):
    c_id = lax.axis_index("c")
    s_id = lax.axis_index("s")
    wid = s_id * 2 + c_id

    for h in range(2):
        base = wid * BPQ + h * HPQ
        cu = pltpu.async_copy(u_rows.at[pl.ds(base, HPQ)], u_v, sem)
        ci = pltpu.async_copy(i_rows.at[pl.ds(base, HPQ)], i_v, sem)
        cu.wait()
        ci.wait()

        def group(g, carry):
            q16 = g * L + lax.iota(jnp.int32, L)
            acc = jnp.zeros((L,), jnp.float32)
            for j in range(D):
                j16 = jnp.full((L,), j, jnp.int32)
                u = plsc.load_gather(u_v, [q16, j16])
                v = plsc.load_gather(i_v, [q16, j16])
                acc = acc + u * v
            out_v[pl.ds(g * L, L)] = acc
            return carry

        lax.fori_loop(0, HPQ // L, group, 0)
        pltpu.sync_copy(out_v, out_hbm.at[pl.ds(base, HPQ)])


@jax.jit
def kernel(user_item_tuple, user_factors, item_factors):
    ids = user_item_tuple.astype(jnp.int32)
    uid = ids[:, 0]
    iid = ids[:, 1]
    uf_t = user_factors.T
    if_t = item_factors.T
    mesh = plsc.VectorSubcoreMesh(core_axis_name="c", subcore_axis_name="s")
    params = pltpu.CompilerParams(
        needs_layout_passes=False, use_tc_tiling_on_sc=True)

    gather = pl.kernel(
        _gather_body,
        out_type=(jax.ShapeDtypeStruct((ROWS_PAD, 128), jnp.float32),
                  jax.ShapeDtypeStruct((ROWS_PAD, 128), jnp.float32)),
        mesh=mesh,
        scratch_types=[
            pltpu.VMEM((2, PIECE), jnp.int32),
            pltpu.VMEM((B + L,), jnp.int32),
            pltpu.VMEM((B + L,), jnp.int32),
            pltpu.VMEM((2, D, CHU), jnp.float32),
            pltpu.VMEM((4, L, 128), jnp.float32),
            pltpu.SemaphoreType.DMA,
            pltpu.SemaphoreType.DMA,
            pltpu.SemaphoreType.DMA,
            pltpu.SemaphoreType.DMA,
            pltpu.SemaphoreType.DMA,
            pltpu.SemaphoreType.DMA,
            pltpu.SemaphoreType.DMA,
            pltpu.SemaphoreType.DMA,
            pltpu.SemaphoreType.DMA,
            pltpu.SemaphoreType.DMA,
        ],
        compiler_params=params,
    )
    u_rows, i_rows = gather(uid, iid, uf_t, if_t)

    dot = pl.kernel(
        _dot_body,
        out_type=jax.ShapeDtypeStruct((B,), jnp.float32),
        mesh=mesh,
        scratch_types=[
            pltpu.VMEM((HPQ, 128), jnp.float32),
            pltpu.VMEM((HPQ, 128), jnp.float32),
            pltpu.VMEM((HPQ,), jnp.float32),
            pltpu.SemaphoreType.DMA,
        ],
        compiler_params=params,
    )
    return dot(u_rows, i_rows)


# speedup vs baseline: 1.5075x; 1.0288x over previous
import jax
import jax.numpy as jnp
from jax import lax
from jax.experimental import pallas as pl
from jax.experimental.pallas import tpu as pltpu
from jax.experimental.pallas import tpu_sc as plsc

NW = 32
B = 16384
D = 64
L = 16
NU = 1000000
NBLK = (NU + 127) // 128
BPWK = (NBLK + NW - 1) // NW
CHB = 4
CHU = CHB * 128
PBITS = 15
PMASK = (1 << PBITS) - 1
ROWS_PAD = B + NW
NPIECE = 4
PIECE = B // NPIECE


def _scan_table(tab_hbm, ids_row, rows_hbm, wid, gs,
                piece_v, hits_v, ch_v, chunk_v, stage_v,
                sem_ids, sem_chk, sem_sc):
    blk0 = wid * BPWK
    u_lo = blk0 * 128
    nb = jnp.minimum(BPWK, NBLK - blk0)
    u_hi = jnp.minimum(u_lo + nb * 128, NU)
    trash = B + wid

    for pc in range(2):
        pltpu.async_copy(ids_row.at[pl.ds(pc * PIECE, PIECE)],
                         piece_v.at[pc], sem_ids[pc])

    nh = 0
    for pc in range(NPIECE):
        pltpu.make_async_copy(ids_row.at[pl.ds(0, PIECE)],
                              piece_v.at[pc & 1], sem_ids[pc & 1]).wait()

        def scan(g, off, _pc=pc):
            v = piece_v[_pc & 1, pl.ds(g * L, L)]
            p = _pc * PIECE + g * L + lax.iota(jnp.int32, L)
            m = (v >= u_lo) & (v < u_hi)
            pk = ((v - u_lo) << PBITS) | p
            plsc.store_compressed(hits_v.at[pl.ds(off, L)], pk, mask=m)
            return off + plsc.all_reduce_population_count(m)[0]

        nh = lax.fori_loop(0, PIECE // L, scan, nh)
        if pc + 2 < NPIECE:
            pltpu.async_copy(ids_row.at[pl.ds((pc + 2) * PIECE, PIECE)],
                             piece_v.at[pc & 1], sem_ids[pc & 1])

    nch = (nb + CHB - 1) // CHB

    def presence(cf):
        def pres(hv, cnts):
            o = pl.multiple_of(hv * L, L)
            pk = hits_v[pl.ds(o, L)]
            lane_ok = (hv * L + lax.iota(jnp.int32, L)) < nh
            rb = (pk >> (PBITS + 7)) - cf * CHB
            out = []
            for cb in range(CHB):
                m = lane_ok & (rb == cb)
                out.append(cnts[cb] +
                           plsc.all_reduce_population_count(m)[0])
            return tuple(out)

        cnts = lax.fori_loop(0, (nh + L - 1) // L, pres,
                             (jnp.int32(0),) * CHB)
        pm = jnp.int32(0)
        for cb in range(CHB):
            pm = pm | (jnp.where(cnts[cb] > 0, jnp.int32(1),
                                 jnp.int32(0)) << cb)
        return pm

    def fire_chunk(c, slot, pm):
        @pl.when(c < nch)
        def _():
            for cb in range(CHB):
                @pl.when(((pm >> cb) & 1) == 1)
                def _(cb=cb):
                    start = pl.multiple_of(u_lo + c * CHU + cb * 128, 128)
                    pltpu.async_copy(tab_hbm.at[:, pl.ds(start, 128)],
                                     chunk_v.at[slot, cb], sem_chk[0])

    pm0 = presence(0)
    fire_chunk(0, 0, pm0)

    def chunk_body(c, carry):
        gs_c, pm_c = carry
        slot = c & 1
        for cb in range(CHB):
            @pl.when(((pm_c >> cb) & 1) == 1)
            def _(cb=cb):
                pltpu.make_async_copy(tab_hbm.at[:, pl.ds(0, 128)],
                                      chunk_v.at[slot, cb],
                                      sem_chk[0]).wait()
        pm_n = presence(c + 1)
        fire_chunk(c + 1, slot ^ 1, pm_n)

        clo = c * CHU

        def comp(hv, off2):
            o = pl.multiple_of(hv * L, L)
            pk = hits_v[pl.ds(o, L)]
            lane_ok = (hv * L + lax.iota(jnp.int32, L)) < nh
            rel = pk >> PBITS
            m2 = (rel >= clo) & (rel < clo + CHU) & lane_ok
            plsc.store_compressed(ch_v.at[pl.ds(off2, L)], pk, mask=m2)
            return off2 + plsc.all_reduce_population_count(m2)[0]

        nh2 = lax.fori_loop(0, (nh + L - 1) // L, comp, 0)
        pad = jnp.full((L,), (clo << PBITS) | trash, jnp.int32)
        plsc.store_compressed(ch_v.at[pl.ds(nh2, L)], pad,
                              mask=jnp.full((L,), True, jnp.bool_))

        ns = (nh2 + L - 1) >> 4

        def shot(s, gs_s):
            for b in range(4):
                @pl.when(((gs_s & 3) == b) & (gs_s >= 4))
                def _(b=b):
                    pltpu.make_async_copy(rows_hbm.at[pl.ds(0, L)],
                                          stage_v.at[b], sem_sc[b]).wait()

            buf = gs_s & 3
            pk16 = ch_v[pl.ds(pl.multiple_of(s * L, L), L)]
            p16 = pk16 & PMASK
            rel = (pk16 >> PBITS) - clo
            stage_s = stage_v.at[buf]
            rows16 = lax.iota(jnp.int32, L)
            slot16 = jnp.full((L,), slot, jnp.int32)
            blk = rel >> 7
            lane = rel & 127
            for j in range(D):
                j16 = jnp.full((L,), j, jnp.int32)
                val = plsc.load_gather(chunk_v, [slot16, blk, j16, lane])
                plsc.store_scatter(stage_s, [rows16, j16], val)
            for b in range(4):
                @pl.when((gs_s & 3) == b)
                def _(b=b):
                    pltpu.async_copy(stage_v.at[b], rows_hbm.at[p16],
                                     sem_sc[b])
            return gs_s + 1

        gs_c = lax.fori_loop(0, ns, shot, gs_c)
        return (gs_c, pm_n)

    gs, _ = lax.fori_loop(0, nch, chunk_body, (gs, pm0))
    return gs


def _gather_body(uid_hbm, iid_hbm, uf_hbm, if_hbm, u_rows, i_rows,
                 piece_v, hits_v, ch_v, chunk_v, stage_v,
                 sem_ids0, sem_ids1,
                 sem_chk0, sem_chk1, sem_chk2, sem_chk3,
                 sem_sc0, sem_sc1, sem_sc2, sem_sc3):
    c_id = lax.axis_index("c")
    s_id = lax.axis_index("s")
    wid = s_id * 2 + c_id
    sem_ids = (sem_ids0, sem_ids1)
    sem_chk = (sem_chk0, sem_chk1, sem_chk2, sem_chk3)
    sem_sc = (sem_sc0, sem_sc1, sem_sc2, sem_sc3)

    gs = _scan_table(uf_hbm, uid_hbm, u_rows, wid, 0,
                     piece_v, hits_v, ch_v, chunk_v, stage_v,
                     sem_ids, sem_chk, sem_sc)
    gs = _scan_table(if_hbm, iid_hbm, i_rows, wid, gs,
                     piece_v, hits_v, ch_v, chunk_v, stage_v,
                     sem_ids, sem_chk, sem_sc)

    for b in range(4):
        @pl.when(gs > b)
        def _(b=b):
            pltpu.make_async_copy(u_rows.at[pl.ds(0, L)],
                                  stage_v.at[b], sem_sc[b]).wait()


BPQ = B // NW
HPQ = BPQ // 2


def _dot_body(u_rows, i_rows, out_hbm, u_v, i_v, out_v, sem):
    c_id = lax.axis_index("c")
    s_id = lax.axis_index("s")
    wid = s_id * 2 + c_id

    for h in range(2):
        base = wid * BPQ + h * HPQ
        cu = pltpu.async_copy(u_rows.at[pl.ds(base, HPQ)], u_v, sem)
        ci = pltpu.async_copy(i_rows.at[pl.ds(base, HPQ)], i_v, sem)
        cu.wait()
        ci.wait()

        def group(g, carry):
            q16 = g * L + lax.iota(jnp.int32, L)
            acc = jnp.zeros((L,), jnp.float32)
            for j in range(D):
                j16 = jnp.full((L,), j, jnp.int32)
                u = plsc.load_gather(u_v, [q16, j16])
                v = plsc.load_gather(i_v, [q16, j16])
                acc = acc + u * v
            out_v[pl.ds(g * L, L)] = acc
            return carry

        lax.fori_loop(0, HPQ // L, group, 0)
        pltpu.sync_copy(out_v, out_hbm.at[pl.ds(base, HPQ)])


@jax.jit
def kernel(user_item_tuple, user_factors, item_factors):
    ids = user_item_tuple.astype(jnp.int32)
    uid = ids[:, 0]
    iid = ids[:, 1]
    uf_t = user_factors.T
    if_t = item_factors.T
    mesh = plsc.VectorSubcoreMesh(core_axis_name="c", subcore_axis_name="s")
    params = pltpu.CompilerParams(
        needs_layout_passes=False, use_tc_tiling_on_sc=True)

    gather = pl.kernel(
        _gather_body,
        out_type=(jax.ShapeDtypeStruct((ROWS_PAD, 128), jnp.float32),
                  jax.ShapeDtypeStruct((ROWS_PAD, 128), jnp.float32)),
        mesh=mesh,
        scratch_types=[
            pltpu.VMEM((2, PIECE), jnp.int32),
            pltpu.VMEM((B + L,), jnp.int32),
            pltpu.VMEM((B + L,), jnp.int32),
            pltpu.VMEM((2, CHB, D, 128), jnp.float32),
            pltpu.VMEM((4, L, 128), jnp.float32),
            pltpu.SemaphoreType.DMA,
            pltpu.SemaphoreType.DMA,
            pltpu.SemaphoreType.DMA,
            pltpu.SemaphoreType.DMA,
            pltpu.SemaphoreType.DMA,
            pltpu.SemaphoreType.DMA,
            pltpu.SemaphoreType.DMA,
            pltpu.SemaphoreType.DMA,
            pltpu.SemaphoreType.DMA,
            pltpu.SemaphoreType.DMA,
        ],
        compiler_params=params,
    )
    u_rows, i_rows = gather(uid, iid, uf_t, if_t)

    dot = pl.kernel(
        _dot_body,
        out_type=jax.ShapeDtypeStruct((B,), jnp.float32),
        mesh=mesh,
        scratch_types=[
            pltpu.VMEM((HPQ, 128), jnp.float32),
            pltpu.VMEM((HPQ, 128), jnp.float32),
            pltpu.VMEM((HPQ,), jnp.float32),
            pltpu.SemaphoreType.DMA,
        ],
        compiler_params=params,
    )
    return dot(u_rows, i_rows)


# speedup vs baseline: 1.5427x; 1.0234x over previous
import jax
import jax.numpy as jnp
from jax import lax
from jax.experimental import pallas as pl
from jax.experimental.pallas import tpu as pltpu
from jax.experimental.pallas import tpu_sc as plsc

NW = 32
B = 16384
D = 64
L = 16
NU = 1000000
NBLK = (NU + 127) // 128
BPWK = (NBLK + NW - 1) // NW
CHB = 4
CHU = CHB * 128
PBITS = 15
PMASK = (1 << PBITS) - 1
ROWS_PAD = B + NW
NPIECE = 4
PIECE = B // NPIECE


def _scan_table(tab_hbm, ids_row, rows_hbm, wid, gs,
                piece_v, hits_v, ch_v, chunk_v, stage_v,
                sem_ids, sem_chk, sem_sc):
    blk0 = wid * BPWK
    u_lo = blk0 * 128
    nb = jnp.minimum(BPWK, NBLK - blk0)
    u_hi = jnp.minimum(u_lo + nb * 128, NU)
    trash = B + wid

    for pc in range(2):
        pltpu.async_copy(ids_row.at[pl.ds(pc * PIECE, PIECE)],
                         piece_v.at[pc], sem_ids[pc])

    nh = 0
    for pc in range(NPIECE):
        pltpu.make_async_copy(ids_row.at[pl.ds(0, PIECE)],
                              piece_v.at[pc & 1], sem_ids[pc & 1]).wait()

        def scan(g, off, _pc=pc):
            v = piece_v[_pc & 1, pl.ds(g * L, L)]
            p = _pc * PIECE + g * L + lax.iota(jnp.int32, L)
            m = (v >= u_lo) & (v < u_hi)
            pk = ((v - u_lo) << PBITS) | p
            plsc.store_compressed(hits_v.at[pl.ds(off, L)], pk, mask=m)
            return off + plsc.all_reduce_population_count(m)[0]

        nh = lax.fori_loop(0, PIECE // L, scan, nh)
        if pc + 2 < NPIECE:
            pltpu.async_copy(ids_row.at[pl.ds((pc + 2) * PIECE, PIECE)],
                             piece_v.at[pc & 1], sem_ids[pc & 1])

    nch = (nb + CHB - 1) // CHB

    def presence(cf):
        def pres(hv, cnts):
            o = pl.multiple_of(hv * L, L)
            pk = hits_v[pl.ds(o, L)]
            lane_ok = (hv * L + lax.iota(jnp.int32, L)) < nh
            rb = (pk >> (PBITS + 7)) - cf * CHB
            out = []
            for cb in range(CHB):
                m = lane_ok & (rb == cb)
                out.append(cnts[cb] +
                           plsc.all_reduce_population_count(m)[0])
            return tuple(out)

        cnts = lax.fori_loop(0, (nh + L - 1) // L, pres,
                             (jnp.int32(0),) * CHB)
        pm = jnp.int32(0)
        for cb in range(CHB):
            pm = pm | (jnp.where(cnts[cb] > 0, jnp.int32(1),
                                 jnp.int32(0)) << cb)
        return pm

    def fire_chunk(c, slot, pm):
        @pl.when(c < nch)
        def _():
            for cb in range(CHB):
                @pl.when(((pm >> cb) & 1) == 1)
                def _(cb=cb):
                    start = pl.multiple_of(u_lo + c * CHU + cb * 128, 128)
                    pltpu.async_copy(tab_hbm.at[:, pl.ds(start, 128)],
                                     chunk_v.at[slot, cb], sem_chk[0])

    pm0 = presence(0)
    fire_chunk(0, 0, pm0)
    pm1 = presence(1)

    def chunk_body(c, carry):
        gs_c, pm_c, pm_n = carry
        slot = c & 1
        for cb in range(CHB):
            @pl.when(((pm_c >> cb) & 1) == 1)
            def _(cb=cb):
                pltpu.make_async_copy(tab_hbm.at[:, pl.ds(0, 128)],
                                      chunk_v.at[slot, cb],
                                      sem_chk[0]).wait()
        fire_chunk(c + 1, slot ^ 1, pm_n)
        pm_n2 = presence(c + 2)

        clo = c * CHU

        def comp(hv, off2):
            o = pl.multiple_of(hv * L, L)
            pk = hits_v[pl.ds(o, L)]
            lane_ok = (hv * L + lax.iota(jnp.int32, L)) < nh
            rel = pk >> PBITS
            m2 = (rel >= clo) & (rel < clo + CHU) & lane_ok
            plsc.store_compressed(ch_v.at[pl.ds(off2, L)], pk, mask=m2)
            return off2 + plsc.all_reduce_population_count(m2)[0]

        nh2 = lax.fori_loop(0, (nh + L - 1) // L, comp, 0)
        pad = jnp.full((L,), (clo << PBITS) | trash, jnp.int32)
        plsc.store_compressed(ch_v.at[pl.ds(nh2, L)], pad,
                              mask=jnp.full((L,), True, jnp.bool_))

        ns = (nh2 + L - 1) >> 4

        def shot(s, gs_s):
            for b in range(4):
                @pl.when(((gs_s & 3) == b) & (gs_s >= 4))
                def _(b=b):
                    pltpu.make_async_copy(rows_hbm.at[pl.ds(0, L)],
                                          stage_v.at[b], sem_sc[b]).wait()

            buf = gs_s & 3
            pk16 = ch_v[pl.ds(pl.multiple_of(s * L, L), L)]
            p16 = pk16 & PMASK
            rel = (pk16 >> PBITS) - clo
            stage_s = stage_v.at[buf]
            rows16 = lax.iota(jnp.int32, L)
            slot16 = jnp.full((L,), slot, jnp.int32)
            blk = rel >> 7
            lane = rel & 127
            for j in range(D):
                j16 = jnp.full((L,), j, jnp.int32)
                val = plsc.load_gather(chunk_v, [slot16, blk, j16, lane])
                plsc.store_scatter(stage_s, [rows16, j16], val)
            for b in range(4):
                @pl.when((gs_s & 3) == b)
                def _(b=b):
                    pltpu.async_copy(stage_v.at[b], rows_hbm.at[p16],
                                     sem_sc[b])
            return gs_s + 1

        gs_c = lax.fori_loop(0, ns, shot, gs_c)
        return (gs_c, pm_n, pm_n2)

    gs, _, _ = lax.fori_loop(0, nch, chunk_body, (gs, pm0, pm1))
    return gs


def _gather_body(uid_hbm, iid_hbm, uf_hbm, if_hbm, u_rows, i_rows,
                 piece_v, hits_v, ch_v, chunk_v, stage_v,
                 sem_ids0, sem_ids1,
                 sem_chk0, sem_chk1, sem_chk2, sem_chk3,
                 sem_sc0, sem_sc1, sem_sc2, sem_sc3):
    c_id = lax.axis_index("c")
    s_id = lax.axis_index("s")
    wid = s_id * 2 + c_id
    sem_ids = (sem_ids0, sem_ids1)
    sem_chk = (sem_chk0, sem_chk1, sem_chk2, sem_chk3)
    sem_sc = (sem_sc0, sem_sc1, sem_sc2, sem_sc3)

    gs = _scan_table(uf_hbm, uid_hbm, u_rows, wid, 0,
                     piece_v, hits_v, ch_v, chunk_v, stage_v,
                     sem_ids, sem_chk, sem_sc)
    gs = _scan_table(if_hbm, iid_hbm, i_rows, wid, gs,
                     piece_v, hits_v, ch_v, chunk_v, stage_v,
                     sem_ids, sem_chk, sem_sc)

    for b in range(4):
        @pl.when(gs > b)
        def _(b=b):
            pltpu.make_async_copy(u_rows.at[pl.ds(0, L)],
                                  stage_v.at[b], sem_sc[b]).wait()


BPQ = B // NW
HPQ = BPQ // 2


def _dot_body(u_rows, i_rows, out_hbm, u_v, i_v, out_v, sem):
    c_id = lax.axis_index("c")
    s_id = lax.axis_index("s")
    wid = s_id * 2 + c_id

    for h in range(2):
        base = wid * BPQ + h * HPQ
        cu = pltpu.async_copy(u_rows.at[pl.ds(base, HPQ)], u_v, sem)
        ci = pltpu.async_copy(i_rows.at[pl.ds(base, HPQ)], i_v, sem)
        cu.wait()
        ci.wait()

        def group(g, carry):
            q16 = g * L + lax.iota(jnp.int32, L)
            acc = jnp.zeros((L,), jnp.float32)
            for j in range(D):
                j16 = jnp.full((L,), j, jnp.int32)
                u = plsc.load_gather(u_v, [q16, j16])
                v = plsc.load_gather(i_v, [q16, j16])
                acc = acc + u * v
            out_v[pl.ds(g * L, L)] = acc
            return carry

        lax.fori_loop(0, HPQ // L, group, 0)
        pltpu.sync_copy(out_v, out_hbm.at[pl.ds(base, HPQ)])


@jax.jit
def kernel(user_item_tuple, user_factors, item_factors):
    ids = user_item_tuple.astype(jnp.int32)
    uid = ids[:, 0]
    iid = ids[:, 1]
    uf_t = user_factors.T
    if_t = item_factors.T
    mesh = plsc.VectorSubcoreMesh(core_axis_name="c", subcore_axis_name="s")
    params = pltpu.CompilerParams(
        needs_layout_passes=False, use_tc_tiling_on_sc=True)

    gather = pl.kernel(
        _gather_body,
        out_type=(jax.ShapeDtypeStruct((ROWS_PAD, 128), jnp.float32),
                  jax.ShapeDtypeStruct((ROWS_PAD, 128), jnp.float32)),
        mesh=mesh,
        scratch_types=[
            pltpu.VMEM((2, PIECE), jnp.int32),
            pltpu.VMEM((B + L,), jnp.int32),
            pltpu.VMEM((B + L,), jnp.int32),
            pltpu.VMEM((2, CHB, D, 128), jnp.float32),
            pltpu.VMEM((4, L, 128), jnp.float32),
            pltpu.SemaphoreType.DMA,
            pltpu.SemaphoreType.DMA,
            pltpu.SemaphoreType.DMA,
            pltpu.SemaphoreType.DMA,
            pltpu.SemaphoreType.DMA,
            pltpu.SemaphoreType.DMA,
            pltpu.SemaphoreType.DMA,
            pltpu.SemaphoreType.DMA,
            pltpu.SemaphoreType.DMA,
            pltpu.SemaphoreType.DMA,
        ],
        compiler_params=params,
    )
    u_rows, i_rows = gather(uid, iid, uf_t, if_t)

    dot = pl.kernel(
        _dot_body,
        out_type=jax.ShapeDtypeStruct((B,), jnp.float32),
        mesh=mesh,
        scratch_types=[
            pltpu.VMEM((HPQ, 128), jnp.float32),
            pltpu.VMEM((HPQ, 128), jnp.float32),
            pltpu.VMEM((HPQ,), jnp.float32),
            pltpu.SemaphoreType.DMA,
        ],
        compiler_params=params,
    )
    return dot(u_rows, i_rows)
